# baseline (speedup 1.0000x reference)
import functools
import jax
import jax.numpy as jnp
from jax import lax
from jax.experimental import pallas as pl
from jax.experimental.pallas import tpu as pltpu
from jax.experimental.pallas import tpu_sc as plsc

BATCH, SEQ, GSIZE = 4096, 50, 8
D_SP, D_AB, D_IT, D_MV = 32, 16, 16, 16
D_OUT = D_SP + D_AB + D_IT + D_MV + 4

_info = plsc.get_sparse_core_info()
NC, NS, L = _info.num_cores, _info.num_subcores, _info.num_lanes
NW = NC * NS
BW = BATCH // NW

_DIMS = (16, 16, 16, 16, 16)
_OFFS = (0, 16, 32, 48, 64)
_TOFF = (200000, 300000, 400000)


def _make_kernel():
    mesh = plsc.VectorSubcoreMesh(core_axis_name="c", subcore_axis_name="s")

    scratch = []
    for _ in range(2):
        scratch.append(pltpu.VMEM((GSIZE, BW), jnp.float32))
        scratch.extend(pltpu.VMEM((BW,), jnp.int32) for _ in range(5))
        scratch.extend(pltpu.VMEM((BW, d), jnp.float32) for d in _DIMS)
        scratch.append(pltpu.VMEM((11, 8, BW), jnp.float32))
    scratch.extend(pltpu.SemaphoreType.DMA for _ in range(6))

    @functools.partial(
        pl.kernel,
        mesh=mesh,
        out_type=jax.ShapeDtypeStruct((SEQ, 11, NW, 8, BW), jnp.float32),
        compiler_params=pltpu.CompilerParams(
            needs_layout_passes=False, use_tc_tiling_on_sc=False),
        scratch_types=scratch,
    )
    def k(x_hbm, tbl_hbm, out_hbm, *s):
        x_v = (s[0], s[12])
        idx = (s[1:6], s[13:18])
        gat = (s[6:11], s[18:23])
        out_v = (s[11], s[23])
        xsem = (s[24], s[25])
        gsem = (s[26], s[27])
        osem = (s[28], s[29])

        wid = lax.axis_index("s") * NC + lax.axis_index("c")
        b0 = wid * BW
        lane = lax.iota(jnp.int32, L)

        def pre(s_, p):
            pltpu.async_copy(x_hbm.at[s_, wid], x_v[p], xsem[p])

        def stage_a(s_, p):
            pltpu.make_async_copy(x_hbm.at[0, 0], x_v[p], xsem[p]).wait()
            for kk in range(BW // L):
                sl = pl.ds(kk * L, L)
                ids0 = jnp.maximum(x_v[p][0, sl].astype(jnp.int32), 0) * 2
                idx[p][0][sl] = ids0
                idx[p][1][sl] = ids0 + 1
                for t in range(3):
                    idx[p][t + 2][sl] = jnp.maximum(
                        x_v[p][t + 1, sl].astype(jnp.int32), 0) + _TOFF[t]
            for t in range(5):
                pltpu.async_copy(tbl_hbm.at[idx[p][t]], gat[p][t], gsem[p])
            @pl.when(s_ >= 2)
            def _():
                pltpu.make_async_copy(
                    out_v[p], out_hbm.at[0, :, 0], osem[p]).wait()
            for c in range(4):
                for kk in range(BW // L):
                    out_v[p][10, c, pl.ds(kk * L, L)] = \
                        x_v[p][4 + c, pl.ds(kk * L, L)]

        def stage_b(s_, p):
            for t in range(5):
                pltpu.make_async_copy(
                    tbl_hbm.at[pl.ds(0, BW)], gat[p][t], gsem[p]).wait()

            def asm(kk, c):
                rows = lane + kk * L
                col = kk * L
                for t in range(5):
                    d = _DIMS[t]
                    vals = [plsc.load_gather(
                                gat[p][t],
                                [rows, jnp.full((L,), f, jnp.int32)])
                            for f in range(d)]
                    for f in range(d):
                        off = _OFFS[t] + f
                        out_v[p][off // 8, off % 8, pl.ds(col, L)] = vals[f]
                return c

            lax.fori_loop(0, BW // L, asm, 0)
            pltpu.async_copy(out_v[p], out_hbm.at[s_, :, wid], osem[p])

        pre(0, 0)
        pre(1, 1)
        stage_a(0, 0)

        def pair(g, c):
            k2 = 2 * g
            for d in range(2):
                pre(k2 + d + 2, d & 1)
                stage_a(k2 + d + 1, (d + 1) & 1)
                stage_b(k2 + d, d & 1)
            return c

        lax.fori_loop(0, (SEQ - 2) // 2, pair, 0)

        stage_a(SEQ - 1, 1)
        stage_b(SEQ - 2, 0)
        stage_b(SEQ - 1, 1)
        for q in range(2):
            pltpu.make_async_copy(
                out_v[q], out_hbm.at[0, :, 0], osem[q]).wait()

    return k


_sc_lookup = _make_kernel()


def kernel(x, species_emb, ability_emb, item_emb, move_emb, group_idx):
    x4 = jnp.transpose(
        jnp.transpose(x, (1, 2, 0)).reshape(SEQ, GSIZE, NW, BW), (0, 2, 1, 3))
    big = jnp.concatenate([species_emb.reshape(200000, 16), ability_emb,
                           item_emb, move_emb], axis=0)
    res = _sc_lookup(x4, big)
    out = jnp.transpose(res, (2, 4, 0, 1, 3)).reshape(BATCH, SEQ, 88)
    return out[:, :, :D_OUT]


# speedup vs baseline: 1.7726x; 1.7726x over previous
import functools
import jax
import jax.numpy as jnp
from jax import lax
from jax.experimental import pallas as pl
from jax.experimental.pallas import tpu as pltpu
from jax.experimental.pallas import tpu_sc as plsc

BATCH, SEQ, GSIZE = 4096, 50, 8
D_SP, D_AB, D_IT, D_MV = 32, 16, 16, 16
D_OUT = D_SP + D_AB + D_IT + D_MV + 4

_info = plsc.get_sparse_core_info()
NC, NS, L = _info.num_cores, _info.num_subcores, _info.num_lanes
NW = NC * NS
BW = BATCH // NW

_DIMS = (D_SP, D_AB, D_IT, D_MV)
_OFFS = (0, D_SP, D_SP + D_AB, D_SP + D_AB + D_IT)


def _make_kernel():
    mesh = plsc.VectorSubcoreMesh(core_axis_name="c", subcore_axis_name="s")

    scratch = []
    for _ in range(2):
        scratch.append(pltpu.VMEM((GSIZE, BW), jnp.float32))
        scratch.extend(pltpu.VMEM((BW,), jnp.int32) for _ in range(4))
        scratch.extend(pltpu.VMEM((BW, d), jnp.float32) for d in _DIMS)
        scratch.append(pltpu.VMEM((11, 8, BW), jnp.float32))
    scratch.extend(pltpu.SemaphoreType.DMA for _ in range(6))

    @functools.partial(
        pl.kernel,
        mesh=mesh,
        out_type=jax.ShapeDtypeStruct((SEQ, 11, NW, 8, BW), jnp.float32),
        compiler_params=pltpu.CompilerParams(
            needs_layout_passes=False, use_tc_tiling_on_sc=False),
        scratch_types=scratch,
    )
    def k(x_hbm, sp_hbm, ab_hbm, it_hbm, mv_hbm, out_hbm, *s):
        x_v = (s[0], s[10])
        idx = (s[1:5], s[11:15])
        gat = (s[5:9], s[15:19])
        out_v = (s[9], s[19])
        xsem = (s[20], s[21])
        gsem = (s[22], s[23])
        osem = (s[24], s[25])
        tables = (sp_hbm, ab_hbm, it_hbm, mv_hbm)

        wid = lax.axis_index("s") * NC + lax.axis_index("c")
        b0 = wid * BW
        lane = lax.iota(jnp.int32, L)

        def pre(s_, p):
            pltpu.async_copy(x_hbm.at[s_, wid], x_v[p], xsem[p])

        def stage_a(s_, p):
            pltpu.make_async_copy(x_hbm.at[0, 0], x_v[p], xsem[p]).wait()
            for t in range(4):
                for kk in range(BW // L):
                    vals = x_v[p][t, pl.ds(kk * L, L)]
                    idx[p][t][pl.ds(kk * L, L)] = jnp.maximum(
                        vals.astype(jnp.int32), 0)
            for t in range(4):
                pltpu.async_copy(tables[t].at[idx[p][t]], gat[p][t], gsem[p])
            @pl.when(s_ >= 2)
            def _():
                pltpu.make_async_copy(
                    out_v[p], out_hbm.at[0, :, 0], osem[p]).wait()
            for c in range(4):
                for kk in range(BW // L):
                    out_v[p][10, c, pl.ds(kk * L, L)] = \
                        x_v[p][4 + c, pl.ds(kk * L, L)]

        def stage_b(s_, p):
            for t in range(4):
                pltpu.make_async_copy(
                    tables[t].at[pl.ds(0, BW)], gat[p][t], gsem[p]).wait()

            def asm(kk, c):
                rows = lane + kk * L
                col = kk * L
                for t in range(4):
                    d = _DIMS[t]
                    vals = [plsc.load_gather(
                                gat[p][t],
                                [rows, jnp.full((L,), f, jnp.int32)])
                            for f in range(d)]
                    for f in range(d):
                        off = _OFFS[t] + f
                        out_v[p][off // 8, off % 8, pl.ds(col, L)] = vals[f]
                return c

            lax.fori_loop(0, BW // L, asm, 0)
            pltpu.async_copy(out_v[p], out_hbm.at[s_, :, wid], osem[p])

        pre(0, 0)
        pre(1, 1)
        stage_a(0, 0)

        def pair(g, c):
            k2 = 2 * g
            for d in range(2):
                pre(k2 + d + 2, d & 1)
                stage_a(k2 + d + 1, (d + 1) & 1)
                stage_b(k2 + d, d & 1)
            return c

        lax.fori_loop(0, (SEQ - 2) // 2, pair, 0)

        stage_a(SEQ - 1, 1)
        stage_b(SEQ - 2, 0)
        stage_b(SEQ - 1, 1)
        for q in range(2):
            pltpu.make_async_copy(
                out_v[q], out_hbm.at[0, :, 0], osem[q]).wait()

    return k


_sc_lookup = _make_kernel()


def kernel(x, species_emb, ability_emb, item_emb, move_emb, group_idx):
    x4 = jnp.transpose(
        jnp.transpose(x, (1, 2, 0)).reshape(SEQ, GSIZE, NW, BW), (0, 2, 1, 3))
    res = _sc_lookup(x4, species_emb, ability_emb, item_emb, move_emb)
    out = jnp.transpose(res, (2, 4, 0, 1, 3)).reshape(BATCH, SEQ, 88)
    return out[:, :, :D_OUT]
